# Initial kernel scaffold; baseline (speedup 1.0000x reference)
import jax
import jax.numpy as jnp
from jax.experimental import pallas as pl


def kernel(x, edge_index, batch, edge_attr, nnW1, nnb1, nnW2, nnb2, rootW, convb, bn_gamma, bn_beta, mlpW1, mlpb1, mlpW2, mlpb2, mlpW3, mlpb3):
    raise NotImplementedError("write your pallas kernel here")


# speedup vs baseline: 6.3123x; 6.3123x over previous
import functools

import jax
import jax.numpy as jnp
from jax import lax
from jax.experimental import pallas as pl
from jax.experimental.pallas import tpu as pltpu
from jax.experimental.pallas import tpu_sc as plsc

N = 10000
E = 160000
D = 16
HID = 16
NUM_LAYERS = 4
NUM_GRAPHS = 64

NW = 32
CHUNK = 128
CPW = 40
EPW = CHUNK * CPW
E_PAD = NW * EPW
NP8 = N // 8
EP8 = E_PAD // 8
EPW8 = EPW // 8
ACC_ROWS = N + 16

HP_STRIPE = 79
OUT_STRIPE = 632

_SC_PARAMS = pltpu.CompilerParams(use_tc_tiling_on_sc=False)


@functools.cache
def _mesh():
    return plsc.VectorSubcoreMesh(core_axis_name="c", subcore_axis_name="s",
                                  num_cores=2, num_subcores=16)


def _gather_body(h_hbm, src_hbm, out_hbm, idx_v, slice_v, nrw_v, chunk_v, pk_v,
                 h_sh):
    cid = lax.axis_index("c")
    sid = lax.axis_index("s")
    wid = sid * 2 + cid
    start = jnp.minimum(sid * HP_STRIPE, NP8 - HP_STRIPE)
    pltpu.sync_copy(h_hbm.at[pl.ds(start, HP_STRIPE)], slice_v)

    def repack_h(q, c):
        for m in range(8):
            nrw_v[q * 8 + m, :] = slice_v[q, pl.ds(m * 16, 16)]
        return c
    lax.fori_loop(0, HP_STRIPE, repack_h, 0)
    pltpu.sync_copy(nrw_v, h_sh.at[pl.ds(start * 8, 8 * HP_STRIPE)])
    pltpu.sync_copy(src_hbm.at[pl.ds(wid * CPW, CPW)], idx_v)
    plsc.subcore_barrier()

    def gather(j, c):
        pltpu.sync_copy(h_sh.at[idx_v.at[j]], chunk_v)
        for mm in range(16):
            for m2 in range(8):
                pk_v[j * 16 + mm, pl.ds(m2 * 16, 16)] = chunk_v[mm * 8 + m2, :]
        return c
    lax.fori_loop(0, CPW, gather, 0)
    pltpu.sync_copy(pk_v, out_hbm.at[pl.ds(wid * EPW8, EPW8)])


@functools.cache
def _gather_sc():
    return pl.kernel(
        _gather_body,
        out_type=jax.ShapeDtypeStruct((EP8, 128), jnp.float32),
        mesh=_mesh(),
        scratch_types=[
            pltpu.VMEM((CPW, CHUNK), jnp.int32),
            pltpu.VMEM((HP_STRIPE, 128), jnp.float32),
            pltpu.VMEM((8 * HP_STRIPE, D), jnp.float32),
            pltpu.VMEM((CHUNK, D), jnp.float32),
            pltpu.VMEM((EPW8, 128), jnp.float32),
            pltpu.VMEM_SHARED((N, D), jnp.float32),
        ],
        compiler_params=_SC_PARAMS,
    )


def _acc_prologue(dst_hbm, idx_v, z_v, acc_sh, sid, wid):
    pltpu.sync_copy(dst_hbm.at[pl.ds(wid * CPW, CPW)], idx_v)

    def zero(j, c):
        z_v[j, :] = jnp.zeros((16,), jnp.float32)
        return c
    lax.fori_loop(0, ACC_ROWS // 16, zero, 0)
    pltpu.sync_copy(z_v, acc_sh.at[pl.ds(sid * (ACC_ROWS // 16), ACC_ROWS // 16)])


def _acc_epilogue(out_hbm, acc_sh, nrw2_v, pk2_v, cid, sid):
    nstart = jnp.minimum(sid * OUT_STRIPE, N - OUT_STRIPE)
    pltpu.sync_copy(acc_sh.at[pl.ds(nstart, OUT_STRIPE)], nrw2_v)

    def repack(q, c):
        for m in range(8):
            pk2_v[q, pl.ds(m * 16, 16)] = nrw2_v[q * 8 + m, :]
        return c
    lax.fori_loop(0, OUT_STRIPE // 8, repack, 0)
    pltpu.sync_copy(pk2_v, out_hbm.at[pl.ds(cid * NP8 + nstart // 8, OUT_STRIPE // 8)])


def _scatter_body(msg_hbm, dst_hbm, out_hbm, idx_v, pk_v, chunk_v, z_v, nrw2_v,
                  pk2_v, acc_sh):
    cid = lax.axis_index("c")
    sid = lax.axis_index("s")
    wid = sid * 2 + cid
    pltpu.sync_copy(msg_hbm.at[pl.ds(wid * EPW8, EPW8)], pk_v)
    _acc_prologue(dst_hbm, idx_v, z_v, acc_sh, sid, wid)
    plsc.subcore_barrier()

    def scat(j, c):
        for mm in range(16):
            for m2 in range(8):
                chunk_v[mm * 8 + m2, :] = pk_v[j * 16 + mm, pl.ds(m2 * 16, 16)]
        pltpu.sync_copy(chunk_v, acc_sh.at[idx_v.at[j]], add=True)
        return c
    lax.fori_loop(0, CPW, scat, 0)
    plsc.subcore_barrier()
    _acc_epilogue(out_hbm, acc_sh, nrw2_v, pk2_v, cid, sid)


@functools.cache
def _scatter_sc():
    return pl.kernel(
        _scatter_body,
        out_type=jax.ShapeDtypeStruct((2 * NP8, 128), jnp.float32),
        mesh=_mesh(),
        scratch_types=[
            pltpu.VMEM((CPW, CHUNK), jnp.int32),
            pltpu.VMEM((EPW8, 128), jnp.float32),
            pltpu.VMEM((CHUNK, D), jnp.float32),
            pltpu.VMEM((ACC_ROWS // 16, D), jnp.float32),
            pltpu.VMEM((OUT_STRIPE, D), jnp.float32),
            pltpu.VMEM((OUT_STRIPE // 8, 128), jnp.float32),
            pltpu.VMEM_SHARED((ACC_ROWS, D), jnp.float32),
        ],
        compiler_params=_SC_PARAMS,
    )


def _degree_body(dst_hbm, out_hbm, idx_v, ones_v, z_v, nrw2_v, pk2_v, acc_sh):
    cid = lax.axis_index("c")
    sid = lax.axis_index("s")
    wid = sid * 2 + cid

    def fill(j, c):
        ones_v[j, :] = jnp.full((16,), 1.0, jnp.float32)
        return c
    lax.fori_loop(0, CHUNK, fill, 0)
    _acc_prologue(dst_hbm, idx_v, z_v, acc_sh, sid, wid)
    plsc.subcore_barrier()

    def scat(j, c):
        pltpu.sync_copy(ones_v, acc_sh.at[idx_v.at[j]], add=True)
        return c
    lax.fori_loop(0, CPW, scat, 0)
    plsc.subcore_barrier()
    _acc_epilogue(out_hbm, acc_sh, nrw2_v, pk2_v, cid, sid)


@functools.cache
def _degree_sc():
    return pl.kernel(
        _degree_body,
        out_type=jax.ShapeDtypeStruct((2 * NP8, 128), jnp.float32),
        mesh=_mesh(),
        scratch_types=[
            pltpu.VMEM((CPW, CHUNK), jnp.int32),
            pltpu.VMEM((CHUNK, D), jnp.float32),
            pltpu.VMEM((ACC_ROWS // 16, D), jnp.float32),
            pltpu.VMEM((OUT_STRIPE, D), jnp.float32),
            pltpu.VMEM((OUT_STRIPE // 8, 128), jnp.float32),
            pltpu.VMEM_SHARED((ACC_ROWS, D), jnp.float32),
        ],
        compiler_params=_SC_PARAMS,
    )


EBP = 512


def _bilinear_body(hs_ref, ea_ref, w1_ref, b1_ref, bdt_ref, rcat_ref, b2_ref,
                   out_ref):
    hs = hs_ref[...]
    hid = jnp.maximum(
        jnp.dot(ea_ref[...], w1_ref[...], preferred_element_type=jnp.float32)
        + b1_ref[...], 0.0)
    gf = jnp.dot(hs, bdt_ref[...], preferred_element_type=jnp.float32)
    hc = jnp.dot(hid, rcat_ref[...], preferred_element_type=jnp.float32)
    p = gf * hc
    msg = jnp.dot(hs, b2_ref[...], preferred_element_type=jnp.float32)
    for j in range(HID):
        msg = msg + p[:, j * 128:(j + 1) * 128]
    out_ref[...] = msg


def _bilinear_tc(hs, ea, w1b, b1t, bdt, rcat, b2b):
    grid = EP8 // EBP
    full = lambda r, c: pl.BlockSpec((r, c), lambda i: (0, 0))
    return pl.pallas_call(
        _bilinear_body,
        grid=(grid,),
        in_specs=[
            pl.BlockSpec((EBP, 128), lambda i: (i, 0)),
            pl.BlockSpec((EBP, 128), lambda i: (i, 0)),
            full(128, 128), full(1, 128), full(128, 2048), full(128, 2048),
            full(128, 128),
        ],
        out_specs=pl.BlockSpec((EBP, 128), lambda i: (i, 0)),
        out_shape=jax.ShapeDtypeStruct((EP8, 128), jnp.float32),
    )(hs, ea, w1b, b1t, bdt, rcat, b2b)


def _group_fold(v128):
    v16 = v128[:, 0:16]
    for g in range(1, 8):
        v16 = v16 + v128[:, g * 16:(g + 1) * 16]
    v16 = v16 * 0.125
    return jnp.concatenate([v16] * 8, axis=1)


def _node_body(aggp_ref, degp_ref, h_ref, rw_ref, cb_ref, gm_ref, bt_ref, out_ref):
    h = h_ref[...]
    deg = jnp.maximum(degp_ref[0:NP8] + degp_ref[NP8:2 * NP8], 1.0)
    agg = (aggp_ref[0:NP8] + aggp_ref[NP8:2 * NP8]) / deg
    out = agg + jnp.dot(h, rw_ref[...], preferred_element_type=jnp.float32) + cb_ref[...]
    mt = _group_fold(jnp.mean(out, axis=0, keepdims=True))
    c = out - mt
    vt = _group_fold(jnp.mean(c * c, axis=0, keepdims=True))
    out = c * lax.rsqrt(vt + 1e-5) * gm_ref[...] + bt_ref[...]
    out_ref[...] = jnp.maximum(out, 0.0) + h


def _node_tc(aggp, degp, h, rwb, cbt, gmt, btt):
    return pl.pallas_call(
        _node_body,
        out_shape=jax.ShapeDtypeStruct((NP8, 128), jnp.float32),
    )(aggp, degp, h, rwb, cbt, gmt, btt)


def _pool_body(h_ref, batch_ref, w1_ref, b1_ref, w2_ref, b2_ref, w3_ref, b3_ref,
               out_ref):
    gid = lax.broadcasted_iota(jnp.int32, (NUM_GRAPHS, NP8), 0)
    sums = jnp.zeros((NUM_GRAPHS, HID), jnp.float32)
    cnt = jnp.zeros((NUM_GRAPHS, 1), jnp.float32)
    for m in range(8):
        onehot = jnp.where(batch_ref[m:m + 1, :] == gid, 1.0, 0.0)
        sums = sums + jnp.dot(onehot, h_ref[:, m * 16:(m + 1) * 16],
                              preferred_element_type=jnp.float32)
        cnt = cnt + jnp.sum(onehot, axis=1, keepdims=True)
    g = sums / jnp.maximum(cnt, 1.0)
    g = jnp.maximum(jnp.dot(g, w1_ref[...], preferred_element_type=jnp.float32)
                    + b1_ref[...], 0.0)
    g = jnp.maximum(jnp.dot(g, w2_ref[...], preferred_element_type=jnp.float32)
                    + b2_ref[...], 0.0)
    out_ref[...] = jnp.dot(g, w3_ref[...], preferred_element_type=jnp.float32) + b3_ref[...]


def _pool_tc(h, batch_pk, w1, b1, w2, b2, w3, b3):
    return pl.pallas_call(
        _pool_body,
        out_shape=jax.ShapeDtypeStruct((NUM_GRAPHS, 1), jnp.float32),
    )(h, batch_pk, w1, b1, w2, b2, w3, b3)


def kernel(x, edge_index, batch, edge_attr, nnW1, nnb1, nnW2, nnb2, rootW, convb,
           bn_gamma, bn_beta, mlpW1, mlpb1, mlpW2, mlpb2, mlpW3, mlpb3):
    src = edge_index[0].astype(jnp.int32)
    dst = edge_index[1].astype(jnp.int32)
    pad = E_PAD - E
    src2d = jnp.concatenate([src, jnp.zeros((pad,), jnp.int32)]).reshape(E_PAD // CHUNK, CHUNK)
    dst2d = jnp.concatenate([dst, jnp.full((pad,), N, jnp.int32)]).reshape(E_PAD // CHUNK, CHUNK)
    ea_pk = jnp.concatenate([edge_attr, jnp.zeros((pad, D), jnp.float32)],
                            axis=0).reshape(EP8, 128)
    x_pk = x.reshape(NP8, 128)
    batch_pk = batch.astype(jnp.int32).reshape(NP8, 8).T

    eye8 = jnp.eye(8, dtype=jnp.float32)
    t_all = nnW2.reshape(NUM_LAYERS, HID, D, HID)
    bdt = jnp.einsum('kjio,lp->klijpo', t_all, eye8).reshape(NUM_LAYERS, 128, 2048)
    rcat = jnp.einsum('aj,lp,o->lajpo', jnp.eye(16, dtype=jnp.float32), eye8,
                      jnp.ones((16,), jnp.float32)).reshape(128, 2048)
    w1b = jnp.einsum('kio,lp->klipo', nnW1, eye8).reshape(NUM_LAYERS, 128, 128)
    b2b = jnp.einsum('kio,lp->klipo', nnb2.reshape(NUM_LAYERS, D, HID),
                     eye8).reshape(NUM_LAYERS, 128, 128)
    rwb = jnp.einsum('kio,lp->klipo', rootW, eye8).reshape(NUM_LAYERS, 128, 128)
    b1t = jnp.tile(nnb1, (1, 8)).reshape(NUM_LAYERS, 1, 128)
    cbt = jnp.tile(convb, (1, 8)).reshape(NUM_LAYERS, 1, 128)
    gmt = jnp.tile(bn_gamma, (1, 8)).reshape(NUM_LAYERS, 1, 128)
    btt = jnp.tile(bn_beta, (1, 8)).reshape(NUM_LAYERS, 1, 128)

    degp = _degree_sc()(dst2d)

    h = x_pk
    for k in range(NUM_LAYERS):
        hs = _gather_sc()(h, src2d)
        msg = _bilinear_tc(hs, ea_pk, w1b[k], b1t[k], bdt[k], rcat, b2b[k])
        aggp = _scatter_sc()(msg, dst2d)
        h = _node_tc(aggp, degp, h, rwb[k], cbt[k], gmt[k], btt[k])

    return _pool_tc(h, batch_pk, mlpW1, mlpb1.reshape(1, HID), mlpW2,
                    mlpb2.reshape(1, HID), mlpW3, mlpb3.reshape(1, 1))


# speedup vs baseline: 6.9361x; 1.0988x over previous
import functools

import jax
import jax.numpy as jnp
from jax import lax
from jax.experimental import pallas as pl
from jax.experimental.pallas import tpu as pltpu
from jax.experimental.pallas import tpu_sc as plsc

N = 10000
E = 160000
D = 16
HID = 16
NUM_LAYERS = 4
NUM_GRAPHS = 64

NW = 32
CHUNK = 128
CPW = 40
EPW = CHUNK * CPW
E_PAD = NW * EPW
NP8 = N // 8
EP8 = E_PAD // 8
EPW8 = EPW // 8
ACC_ROWS = N + 16

NSLOT = 4
NGRP = CPW // NSLOT

HP_STRIPE = 79
OUT_STRIPE = 632

_SC_PARAMS = pltpu.CompilerParams(use_tc_tiling_on_sc=False)


@functools.cache
def _mesh():
    return plsc.VectorSubcoreMesh(core_axis_name="c", subcore_axis_name="s",
                                  num_cores=2, num_subcores=16)


def _gather_body(h_hbm, src_hbm, out_hbm, idx_v, slice_v, nrw_v, chunk_v, pk_v,
                 h_sh, s0, s1, s2, s3):
    cid = lax.axis_index("c")
    sid = lax.axis_index("s")
    wid = sid * 2 + cid
    sems = (s0, s1, s2, s3)
    start = jnp.minimum(sid * HP_STRIPE, NP8 - HP_STRIPE)
    pltpu.sync_copy(h_hbm.at[pl.ds(start, HP_STRIPE)], slice_v)

    def repack_h(q, c):
        for m in range(8):
            nrw_v[q * 8 + m, :] = slice_v[q, pl.ds(m * 16, 16)]
        return c
    lax.fori_loop(0, HP_STRIPE, repack_h, 0)
    pltpu.sync_copy(nrw_v, h_sh.at[pl.ds(start * 8, 8 * HP_STRIPE)])
    pltpu.sync_copy(src_hbm.at[pl.ds(wid * CPW, CPW)], idx_v)
    plsc.subcore_barrier()

    for b in range(NSLOT):
        pltpu.async_copy(h_sh.at[idx_v.at[b]], chunk_v.at[b], sems[b])

    def group(g, c):
        for b in range(NSLOT):
            j = g * NSLOT + b
            pltpu.make_async_copy(h_sh.at[idx_v.at[j]], chunk_v.at[b],
                                  sems[b]).wait()
            for mm in range(16):
                for m2 in range(8):
                    pk_v[j * 16 + mm, pl.ds(m2 * 16, 16)] = chunk_v[b, mm * 8 + m2, :]

            @pl.when(g < NGRP - 1)
            def _():
                pltpu.async_copy(h_sh.at[idx_v.at[j + NSLOT]], chunk_v.at[b],
                                 sems[b])
        return c
    lax.fori_loop(0, NGRP, group, 0)
    pltpu.sync_copy(pk_v, out_hbm.at[pl.ds(wid * EPW8, EPW8)])


@functools.cache
def _gather_sc():
    return pl.kernel(
        _gather_body,
        out_type=jax.ShapeDtypeStruct((EP8, 128), jnp.float32),
        mesh=_mesh(),
        scratch_types=[
            pltpu.VMEM((CPW, CHUNK), jnp.int32),
            pltpu.VMEM((HP_STRIPE, 128), jnp.float32),
            pltpu.VMEM((8 * HP_STRIPE, D), jnp.float32),
            pltpu.VMEM((NSLOT, CHUNK, D), jnp.float32),
            pltpu.VMEM((EPW8, 128), jnp.float32),
            pltpu.VMEM_SHARED((N, D), jnp.float32),
            pltpu.SemaphoreType.DMA, pltpu.SemaphoreType.DMA,
            pltpu.SemaphoreType.DMA, pltpu.SemaphoreType.DMA,
        ],
        compiler_params=_SC_PARAMS,
    )


def _zero_acc(z_v, acc_sh, sid):
    def zero(j, c):
        z_v[j, :] = jnp.zeros((16,), jnp.float32)
        return c
    lax.fori_loop(0, ACC_ROWS // 16, zero, 0)
    pltpu.sync_copy(z_v, acc_sh.at[pl.ds(sid * (ACC_ROWS // 16), ACC_ROWS // 16)])


def _acc_epilogue(out_hbm, acc_sh, nrw2_v, pk2_v, cid, sid):
    nstart = jnp.minimum(sid * OUT_STRIPE, N - OUT_STRIPE)
    pltpu.sync_copy(acc_sh.at[pl.ds(nstart, OUT_STRIPE)], nrw2_v)

    def repack(q, c):
        for m in range(8):
            pk2_v[q, pl.ds(m * 16, 16)] = nrw2_v[q * 8 + m, :]
        return c
    lax.fori_loop(0, OUT_STRIPE // 8, repack, 0)
    pltpu.sync_copy(pk2_v, out_hbm.at[pl.ds(cid * NP8 + nstart // 8, OUT_STRIPE // 8)])


def _scatter_pipeline(msg_hbm, idx_v, pk_v, chunk_v, acc_sh, wid, lds, sts,
                      per_chunk_extra):
    for b in range(NSLOT):
        pltpu.async_copy(msg_hbm.at[pl.ds(wid * EPW8 + b * 16, 16)], pk_v.at[b],
                         lds[b])

    def group(g, c):
        for b in range(NSLOT):
            j = g * NSLOT + b
            pltpu.make_async_copy(msg_hbm.at[pl.ds(wid * EPW8 + j * 16, 16)],
                                  pk_v.at[b], lds[b]).wait()

            @pl.when(g > 0)
            def _():
                pltpu.make_async_copy(chunk_v.at[b], acc_sh.at[idx_v.at[j]],
                                      sts[b]).wait()
            for mm in range(16):
                for m2 in range(8):
                    chunk_v[b, mm * 8 + m2, :] = pk_v[b, mm, pl.ds(m2 * 16, 16)]

            @pl.when(g < NGRP - 1)
            def _():
                pltpu.async_copy(msg_hbm.at[pl.ds(wid * EPW8 + (j + NSLOT) * 16, 16)],
                                 pk_v.at[b], lds[b])
            pltpu.async_copy(chunk_v.at[b], acc_sh.at[idx_v.at[j]], sts[b],
                             add=True)
            per_chunk_extra(j)
        return c
    lax.fori_loop(0, NGRP, group, 0)
    for b in range(NSLOT):
        pltpu.make_async_copy(chunk_v.at[b], acc_sh.at[idx_v.at[0]], sts[b]).wait()


def _scatter_body(msg_hbm, dst_hbm, out_hbm, idx_v, pk_v, chunk_v, z_v, nrw2_v,
                  pk2_v, acc_sh, l0, l1, l2, l3, t0, t1, t2, t3):
    cid = lax.axis_index("c")
    sid = lax.axis_index("s")
    wid = sid * 2 + cid
    pltpu.sync_copy(dst_hbm.at[pl.ds(wid * CPW, CPW)], idx_v)
    _zero_acc(z_v, acc_sh, sid)
    plsc.subcore_barrier()
    _scatter_pipeline(msg_hbm, idx_v, pk_v, chunk_v, acc_sh, wid,
                      (l0, l1, l2, l3), (t0, t1, t2, t3), lambda j: None)
    plsc.subcore_barrier()
    _acc_epilogue(out_hbm, acc_sh, nrw2_v, pk2_v, cid, sid)


@functools.cache
def _scatter_sc():
    return pl.kernel(
        _scatter_body,
        out_type=jax.ShapeDtypeStruct((2 * NP8, 128), jnp.float32),
        mesh=_mesh(),
        scratch_types=[
            pltpu.VMEM((CPW, CHUNK), jnp.int32),
            pltpu.VMEM((NSLOT, 16, 128), jnp.float32),
            pltpu.VMEM((NSLOT, CHUNK, D), jnp.float32),
            pltpu.VMEM((ACC_ROWS // 16, D), jnp.float32),
            pltpu.VMEM((OUT_STRIPE, D), jnp.float32),
            pltpu.VMEM((OUT_STRIPE // 8, 128), jnp.float32),
            pltpu.VMEM_SHARED((ACC_ROWS, D), jnp.float32),
        ] + [pltpu.SemaphoreType.DMA] * (2 * NSLOT),
        compiler_params=_SC_PARAMS,
    )


def _scatter_deg_body(msg_hbm, dst_hbm, out_hbm, deg_hbm, idx_v, pk_v, chunk_v,
                      z_v, nrw2_v, pk2_v, ones_v, acc_sh, dacc_sh,
                      l0, l1, l2, l3, t0, t1, t2, t3, so):
    cid = lax.axis_index("c")
    sid = lax.axis_index("s")
    wid = sid * 2 + cid
    pltpu.sync_copy(dst_hbm.at[pl.ds(wid * CPW, CPW)], idx_v)
    _zero_acc(z_v, acc_sh, sid)
    _zero_acc(z_v, dacc_sh, sid)

    def fill(j, c):
        ones_v[j, :] = jnp.full((16,), 1.0, jnp.float32)
        return c
    lax.fori_loop(0, CHUNK, fill, 0)
    plsc.subcore_barrier()

    def ones_add(j):
        pltpu.async_copy(ones_v, dacc_sh.at[idx_v.at[j]], so, add=True)

    _scatter_pipeline(msg_hbm, idx_v, pk_v, chunk_v, acc_sh, wid,
                      (l0, l1, l2, l3), (t0, t1, t2, t3), ones_add)

    def drain(j, c):
        pltpu.make_async_copy(ones_v, dacc_sh.at[idx_v.at[0]], so).wait()
        return c
    lax.fori_loop(0, CPW, drain, 0)
    plsc.subcore_barrier()
    _acc_epilogue(out_hbm, acc_sh, nrw2_v, pk2_v, cid, sid)
    _acc_epilogue(deg_hbm, dacc_sh, nrw2_v, pk2_v, cid, sid)


@functools.cache
def _scatter_deg_sc():
    return pl.kernel(
        _scatter_deg_body,
        out_type=(jax.ShapeDtypeStruct((2 * NP8, 128), jnp.float32),
                  jax.ShapeDtypeStruct((2 * NP8, 128), jnp.float32)),
        mesh=_mesh(),
        scratch_types=[
            pltpu.VMEM((CPW, CHUNK), jnp.int32),
            pltpu.VMEM((NSLOT, 16, 128), jnp.float32),
            pltpu.VMEM((NSLOT, CHUNK, D), jnp.float32),
            pltpu.VMEM((ACC_ROWS // 16, D), jnp.float32),
            pltpu.VMEM((OUT_STRIPE, D), jnp.float32),
            pltpu.VMEM((OUT_STRIPE // 8, 128), jnp.float32),
            pltpu.VMEM((CHUNK, D), jnp.float32),
            pltpu.VMEM_SHARED((ACC_ROWS, D), jnp.float32),
            pltpu.VMEM_SHARED((ACC_ROWS, D), jnp.float32),
        ] + [pltpu.SemaphoreType.DMA] * (2 * NSLOT + 1),
        compiler_params=_SC_PARAMS,
    )


EBP = 512


def _bilinear_body(hs_ref, ea_ref, w1_ref, b1_ref, bdt_ref, rcat_ref, b2_ref,
                   out_ref):
    hs = hs_ref[...]
    hid = jnp.maximum(
        jnp.dot(ea_ref[...], w1_ref[...], preferred_element_type=jnp.float32)
        + b1_ref[...], 0.0)
    gf = jnp.dot(hs, bdt_ref[...], preferred_element_type=jnp.float32)
    hc = jnp.dot(hid, rcat_ref[...], preferred_element_type=jnp.float32)
    p = gf * hc
    msg = jnp.dot(hs, b2_ref[...], preferred_element_type=jnp.float32)
    for j in range(HID):
        msg = msg + p[:, j * 128:(j + 1) * 128]
    out_ref[...] = msg


def _bilinear_tc(hs, ea, w1b, b1t, bdt, rcat, b2b):
    grid = EP8 // EBP
    full = lambda r, c: pl.BlockSpec((r, c), lambda i: (0, 0))
    return pl.pallas_call(
        _bilinear_body,
        grid=(grid,),
        in_specs=[
            pl.BlockSpec((EBP, 128), lambda i: (i, 0)),
            pl.BlockSpec((EBP, 128), lambda i: (i, 0)),
            full(128, 128), full(1, 128), full(128, 2048), full(128, 2048),
            full(128, 128),
        ],
        out_specs=pl.BlockSpec((EBP, 128), lambda i: (i, 0)),
        out_shape=jax.ShapeDtypeStruct((EP8, 128), jnp.float32),
    )(hs, ea, w1b, b1t, bdt, rcat, b2b)


def _group_fold(v128):
    v16 = v128[:, 0:16]
    for g in range(1, 8):
        v16 = v16 + v128[:, g * 16:(g + 1) * 16]
    v16 = v16 * 0.125
    return jnp.concatenate([v16] * 8, axis=1)


def _node_body(aggp_ref, degp_ref, h_ref, rw_ref, cb_ref, gm_ref, bt_ref, out_ref):
    h = h_ref[...]
    deg = jnp.maximum(degp_ref[0:NP8] + degp_ref[NP8:2 * NP8], 1.0)
    agg = (aggp_ref[0:NP8] + aggp_ref[NP8:2 * NP8]) / deg
    out = agg + jnp.dot(h, rw_ref[...], preferred_element_type=jnp.float32) + cb_ref[...]
    mt = _group_fold(jnp.mean(out, axis=0, keepdims=True))
    c = out - mt
    vt = _group_fold(jnp.mean(c * c, axis=0, keepdims=True))
    out = c * lax.rsqrt(vt + 1e-5) * gm_ref[...] + bt_ref[...]
    out_ref[...] = jnp.maximum(out, 0.0) + h


def _node_tc(aggp, degp, h, rwb, cbt, gmt, btt):
    return pl.pallas_call(
        _node_body,
        out_shape=jax.ShapeDtypeStruct((NP8, 128), jnp.float32),
    )(aggp, degp, h, rwb, cbt, gmt, btt)


def _pool_body(h_ref, batch_ref, w1_ref, b1_ref, w2_ref, b2_ref, w3_ref, b3_ref,
               out_ref):
    gid = lax.broadcasted_iota(jnp.int32, (NUM_GRAPHS, NP8), 0)
    sums = jnp.zeros((NUM_GRAPHS, HID), jnp.float32)
    cnt = jnp.zeros((NUM_GRAPHS, 1), jnp.float32)
    for m in range(8):
        onehot = jnp.where(batch_ref[m:m + 1, :] == gid, 1.0, 0.0)
        sums = sums + jnp.dot(onehot, h_ref[:, m * 16:(m + 1) * 16],
                              preferred_element_type=jnp.float32)
        cnt = cnt + jnp.sum(onehot, axis=1, keepdims=True)
    g = sums / jnp.maximum(cnt, 1.0)
    g = jnp.maximum(jnp.dot(g, w1_ref[...], preferred_element_type=jnp.float32)
                    + b1_ref[...], 0.0)
    g = jnp.maximum(jnp.dot(g, w2_ref[...], preferred_element_type=jnp.float32)
                    + b2_ref[...], 0.0)
    out_ref[...] = jnp.dot(g, w3_ref[...], preferred_element_type=jnp.float32) + b3_ref[...]


def _pool_tc(h, batch_pk, w1, b1, w2, b2, w3, b3):
    return pl.pallas_call(
        _pool_body,
        out_shape=jax.ShapeDtypeStruct((NUM_GRAPHS, 1), jnp.float32),
    )(h, batch_pk, w1, b1, w2, b2, w3, b3)


def kernel(x, edge_index, batch, edge_attr, nnW1, nnb1, nnW2, nnb2, rootW, convb,
           bn_gamma, bn_beta, mlpW1, mlpb1, mlpW2, mlpb2, mlpW3, mlpb3):
    src = edge_index[0].astype(jnp.int32)
    dst = edge_index[1].astype(jnp.int32)
    pad = E_PAD - E
    src2d = jnp.concatenate([src, jnp.zeros((pad,), jnp.int32)]).reshape(E_PAD // CHUNK, CHUNK)
    dst2d = jnp.concatenate([dst, jnp.full((pad,), N, jnp.int32)]).reshape(E_PAD // CHUNK, CHUNK)
    ea_pk = jnp.concatenate([edge_attr, jnp.zeros((pad, D), jnp.float32)],
                            axis=0).reshape(EP8, 128)
    x_pk = x.reshape(NP8, 128)
    batch_pk = batch.astype(jnp.int32).reshape(NP8, 8).T

    eye8 = jnp.eye(8, dtype=jnp.float32)
    t_all = nnW2.reshape(NUM_LAYERS, HID, D, HID)
    bdt = jnp.einsum('kjio,lp->klijpo', t_all, eye8).reshape(NUM_LAYERS, 128, 2048)
    rcat = jnp.einsum('aj,lp,o->lajpo', jnp.eye(16, dtype=jnp.float32), eye8,
                      jnp.ones((16,), jnp.float32)).reshape(128, 2048)
    w1b = jnp.einsum('kio,lp->klipo', nnW1, eye8).reshape(NUM_LAYERS, 128, 128)
    b2b = jnp.einsum('kio,lp->klipo', nnb2.reshape(NUM_LAYERS, D, HID),
                     eye8).reshape(NUM_LAYERS, 128, 128)
    rwb = jnp.einsum('kio,lp->klipo', rootW, eye8).reshape(NUM_LAYERS, 128, 128)
    b1t = jnp.tile(nnb1, (1, 8)).reshape(NUM_LAYERS, 1, 128)
    cbt = jnp.tile(convb, (1, 8)).reshape(NUM_LAYERS, 1, 128)
    gmt = jnp.tile(bn_gamma, (1, 8)).reshape(NUM_LAYERS, 1, 128)
    btt = jnp.tile(bn_beta, (1, 8)).reshape(NUM_LAYERS, 1, 128)

    h = x_pk
    degp = None
    for k in range(NUM_LAYERS):
        hs = _gather_sc()(h, src2d)
        msg = _bilinear_tc(hs, ea_pk, w1b[k], b1t[k], bdt[k], rcat, b2b[k])
        if k == 0:
            aggp, degp = _scatter_deg_sc()(msg, dst2d)
        else:
            aggp = _scatter_sc()(msg, dst2d)
        h = _node_tc(aggp, degp, h, rwb[k], cbt[k], gmt[k], btt[k])

    return _pool_tc(h, batch_pk, mlpW1, mlpb1.reshape(1, HID), mlpW2,
                    mlpb2.reshape(1, HID), mlpW3, mlpb3.reshape(1, 1))


# speedup vs baseline: 7.0668x; 1.0188x over previous
import functools

import jax
import jax.numpy as jnp
from jax import lax
from jax.experimental import pallas as pl
from jax.experimental.pallas import tpu as pltpu
from jax.experimental.pallas import tpu_sc as plsc

N = 10000
E = 160000
D = 16
HID = 16
NUM_LAYERS = 4
NUM_GRAPHS = 64

NW = 32
CHUNK = 128
CPW = 40
EPW = CHUNK * CPW
E_PAD = NW * EPW
NP8 = N // 8
EP8 = E_PAD // 8
EPW8 = EPW // 8
ACC_ROWS = N + 16

NSLOT = 4
NGRP = CPW // NSLOT

HP_STRIPE = 79
OUT_STRIPE = 632

_SC_PARAMS = pltpu.CompilerParams(use_tc_tiling_on_sc=False)


@functools.cache
def _mesh():
    return plsc.VectorSubcoreMesh(core_axis_name="c", subcore_axis_name="s",
                                  num_cores=2, num_subcores=16)


def _gather_body(h_hbm, src_hbm, out_hbm, idx_v, slice_v, nrw_v, chunk_v, pk_v,
                 h_sh, s0, s1, s2, s3):
    cid = lax.axis_index("c")
    sid = lax.axis_index("s")
    wid = sid * 2 + cid
    sems = (s0, s1, s2, s3)
    start = jnp.minimum(sid * HP_STRIPE, NP8 - HP_STRIPE)
    pltpu.sync_copy(h_hbm.at[pl.ds(start, HP_STRIPE)], slice_v)

    def repack_h(q, c):
        for m in range(8):
            nrw_v[q * 8 + m, :] = slice_v[q, pl.ds(m * 16, 16)]
        return c
    lax.fori_loop(0, HP_STRIPE, repack_h, 0)
    pltpu.sync_copy(nrw_v, h_sh.at[pl.ds(start * 8, 8 * HP_STRIPE)])
    pltpu.sync_copy(src_hbm.at[pl.ds(wid * CPW, CPW)], idx_v)
    plsc.subcore_barrier()

    for b in range(NSLOT):
        pltpu.async_copy(h_sh.at[idx_v.at[b]], chunk_v.at[b], sems[b])

    def group(g, c):
        for b in range(NSLOT):
            j = g * NSLOT + b
            pltpu.make_async_copy(h_sh.at[idx_v.at[j]], chunk_v.at[b],
                                  sems[b]).wait()
            for mm in range(16):
                for m2 in range(8):
                    pk_v[j * 16 + mm, pl.ds(m2 * 16, 16)] = chunk_v[b, mm * 8 + m2, :]

            @pl.when(g < NGRP - 1)
            def _():
                pltpu.async_copy(h_sh.at[idx_v.at[j + NSLOT]], chunk_v.at[b],
                                 sems[b])
        return c
    lax.fori_loop(0, NGRP, group, 0)
    pltpu.sync_copy(pk_v, out_hbm.at[pl.ds(wid * EPW8, EPW8)])


@functools.cache
def _gather_sc():
    return pl.kernel(
        _gather_body,
        out_type=jax.ShapeDtypeStruct((EP8, 128), jnp.float32),
        mesh=_mesh(),
        scratch_types=[
            pltpu.VMEM((CPW, CHUNK), jnp.int32),
            pltpu.VMEM((HP_STRIPE, 128), jnp.float32),
            pltpu.VMEM((8 * HP_STRIPE, D), jnp.float32),
            pltpu.VMEM((NSLOT, CHUNK, D), jnp.float32),
            pltpu.VMEM((EPW8, 128), jnp.float32),
            pltpu.VMEM_SHARED((N, D), jnp.float32),
            pltpu.SemaphoreType.DMA, pltpu.SemaphoreType.DMA,
            pltpu.SemaphoreType.DMA, pltpu.SemaphoreType.DMA,
        ],
        compiler_params=_SC_PARAMS,
    )


def _zero_acc(z_v, acc_sh, sid):
    def zero(j, c):
        z_v[j, :] = jnp.zeros((16,), jnp.float32)
        return c
    lax.fori_loop(0, ACC_ROWS // 16, zero, 0)
    pltpu.sync_copy(z_v, acc_sh.at[pl.ds(sid * (ACC_ROWS // 16), ACC_ROWS // 16)])


def _acc_epilogue(out_hbm, acc_sh, nrw2_v, pk2_v, cid, sid):
    nstart = jnp.minimum(sid * OUT_STRIPE, N - OUT_STRIPE)
    pltpu.sync_copy(acc_sh.at[pl.ds(nstart, OUT_STRIPE)], nrw2_v)

    def repack(q, c):
        for m in range(8):
            pk2_v[q, pl.ds(m * 16, 16)] = nrw2_v[q * 8 + m, :]
        return c
    lax.fori_loop(0, OUT_STRIPE // 8, repack, 0)
    pltpu.sync_copy(pk2_v, out_hbm.at[pl.ds(cid * NP8 + nstart // 8, OUT_STRIPE // 8)])


def _scatter_pipeline(msg_hbm, idx_v, pk_v, chunk_v, acc_sh, wid, lds, sts,
                      per_chunk_extra):
    for b in range(NSLOT):
        pltpu.async_copy(msg_hbm.at[pl.ds(wid * EPW8 + b * 16, 16)], pk_v.at[b],
                         lds[b])

    def group(g, c):
        for b in range(NSLOT):
            j = g * NSLOT + b
            pltpu.make_async_copy(msg_hbm.at[pl.ds(wid * EPW8 + j * 16, 16)],
                                  pk_v.at[b], lds[b]).wait()

            @pl.when(g > 0)
            def _():
                pltpu.make_async_copy(chunk_v.at[b], acc_sh.at[idx_v.at[j]],
                                      sts[b]).wait()
            for mm in range(16):
                for m2 in range(8):
                    chunk_v[b, mm * 8 + m2, :] = pk_v[b, mm, pl.ds(m2 * 16, 16)]

            @pl.when(g < NGRP - 1)
            def _():
                pltpu.async_copy(msg_hbm.at[pl.ds(wid * EPW8 + (j + NSLOT) * 16, 16)],
                                 pk_v.at[b], lds[b])
            pltpu.async_copy(chunk_v.at[b], acc_sh.at[idx_v.at[j]], sts[b],
                             add=True)
            per_chunk_extra(j)
        return c
    lax.fori_loop(0, NGRP, group, 0)
    for b in range(NSLOT):
        pltpu.make_async_copy(chunk_v.at[b], acc_sh.at[idx_v.at[0]], sts[b]).wait()


def _scatter_body(msg_hbm, dst_hbm, out_hbm, idx_v, pk_v, chunk_v, z_v, nrw2_v,
                  pk2_v, acc_sh, l0, l1, l2, l3, t0, t1, t2, t3):
    cid = lax.axis_index("c")
    sid = lax.axis_index("s")
    wid = sid * 2 + cid
    pltpu.sync_copy(dst_hbm.at[pl.ds(wid * CPW, CPW)], idx_v)
    _zero_acc(z_v, acc_sh, sid)
    plsc.subcore_barrier()
    _scatter_pipeline(msg_hbm, idx_v, pk_v, chunk_v, acc_sh, wid,
                      (l0, l1, l2, l3), (t0, t1, t2, t3), lambda j: None)
    plsc.subcore_barrier()
    _acc_epilogue(out_hbm, acc_sh, nrw2_v, pk2_v, cid, sid)


@functools.cache
def _scatter_sc():
    return pl.kernel(
        _scatter_body,
        out_type=jax.ShapeDtypeStruct((2 * NP8, 128), jnp.float32),
        mesh=_mesh(),
        scratch_types=[
            pltpu.VMEM((CPW, CHUNK), jnp.int32),
            pltpu.VMEM((NSLOT, 16, 128), jnp.float32),
            pltpu.VMEM((NSLOT, CHUNK, D), jnp.float32),
            pltpu.VMEM((ACC_ROWS // 16, D), jnp.float32),
            pltpu.VMEM((OUT_STRIPE, D), jnp.float32),
            pltpu.VMEM((OUT_STRIPE // 8, 128), jnp.float32),
            pltpu.VMEM_SHARED((ACC_ROWS, D), jnp.float32),
        ] + [pltpu.SemaphoreType.DMA] * (2 * NSLOT),
        compiler_params=_SC_PARAMS,
    )


def _scatter_deg_body(msg_hbm, dst_hbm, out_hbm, deg_hbm, idx_v, pk_v, chunk_v,
                      z_v, nrw2_v, pk2_v, ones_v, acc_sh, dacc_sh,
                      l0, l1, l2, l3, t0, t1, t2, t3, so):
    cid = lax.axis_index("c")
    sid = lax.axis_index("s")
    wid = sid * 2 + cid
    pltpu.sync_copy(dst_hbm.at[pl.ds(wid * CPW, CPW)], idx_v)
    _zero_acc(z_v, acc_sh, sid)
    _zero_acc(z_v, dacc_sh, sid)

    def fill(j, c):
        ones_v[j, :] = jnp.full((16,), 1.0, jnp.float32)
        return c
    lax.fori_loop(0, CHUNK, fill, 0)
    plsc.subcore_barrier()

    def ones_add(j):
        pltpu.async_copy(ones_v, dacc_sh.at[idx_v.at[j]], so, add=True)

    _scatter_pipeline(msg_hbm, idx_v, pk_v, chunk_v, acc_sh, wid,
                      (l0, l1, l2, l3), (t0, t1, t2, t3), ones_add)

    def drain(j, c):
        pltpu.make_async_copy(ones_v, dacc_sh.at[idx_v.at[0]], so).wait()
        return c
    lax.fori_loop(0, CPW, drain, 0)
    plsc.subcore_barrier()
    _acc_epilogue(out_hbm, acc_sh, nrw2_v, pk2_v, cid, sid)
    _acc_epilogue(deg_hbm, dacc_sh, nrw2_v, pk2_v, cid, sid)


@functools.cache
def _scatter_deg_sc():
    return pl.kernel(
        _scatter_deg_body,
        out_type=(jax.ShapeDtypeStruct((2 * NP8, 128), jnp.float32),
                  jax.ShapeDtypeStruct((2 * NP8, 128), jnp.float32)),
        mesh=_mesh(),
        scratch_types=[
            pltpu.VMEM((CPW, CHUNK), jnp.int32),
            pltpu.VMEM((NSLOT, 16, 128), jnp.float32),
            pltpu.VMEM((NSLOT, CHUNK, D), jnp.float32),
            pltpu.VMEM((ACC_ROWS // 16, D), jnp.float32),
            pltpu.VMEM((OUT_STRIPE, D), jnp.float32),
            pltpu.VMEM((OUT_STRIPE // 8, 128), jnp.float32),
            pltpu.VMEM((CHUNK, D), jnp.float32),
            pltpu.VMEM_SHARED((ACC_ROWS, D), jnp.float32),
            pltpu.VMEM_SHARED((ACC_ROWS, D), jnp.float32),
        ] + [pltpu.SemaphoreType.DMA] * (2 * NSLOT + 1),
        compiler_params=_SC_PARAMS,
    )


EBP = 1024


def _bilinear_body(hs_ref, ea_ref, w1_ref, b1_ref, bdt_ref, rcat_ref, b2_ref,
                   out_ref):
    hs = hs_ref[...]
    hid = jnp.maximum(
        jnp.dot(ea_ref[...], w1_ref[...], preferred_element_type=jnp.float32)
        + b1_ref[...], 0.0)
    gf = jnp.dot(hs, bdt_ref[...], preferred_element_type=jnp.float32)
    hc = jnp.dot(hid, rcat_ref[...], preferred_element_type=jnp.float32)
    p = gf * hc
    msg = jnp.dot(hs, b2_ref[...], preferred_element_type=jnp.float32)
    for j in range(HID):
        msg = msg + p[:, j * 128:(j + 1) * 128]
    out_ref[...] = msg


def _bilinear_tc(hs, ea, w1b, b1t, bdt, rcat, b2b):
    grid = EP8 // EBP
    full = lambda r, c: pl.BlockSpec((r, c), lambda i: (0, 0))
    return pl.pallas_call(
        _bilinear_body,
        grid=(grid,),
        in_specs=[
            pl.BlockSpec((EBP, 128), lambda i: (i, 0)),
            pl.BlockSpec((EBP, 128), lambda i: (i, 0)),
            full(128, 128), full(1, 128), full(128, 2048), full(128, 2048),
            full(128, 128),
        ],
        out_specs=pl.BlockSpec((EBP, 128), lambda i: (i, 0)),
        out_shape=jax.ShapeDtypeStruct((EP8, 128), jnp.float32),
    )(hs, ea, w1b, b1t, bdt, rcat, b2b)


def _group_fold(v128):
    v16 = v128[:, 0:16]
    for g in range(1, 8):
        v16 = v16 + v128[:, g * 16:(g + 1) * 16]
    v16 = v16 * 0.125
    return jnp.concatenate([v16] * 8, axis=1)


def _node_body(aggp_ref, degp_ref, h_ref, rw_ref, cb_ref, gm_ref, bt_ref, out_ref):
    h = h_ref[...]
    deg = jnp.maximum(degp_ref[0:NP8] + degp_ref[NP8:2 * NP8], 1.0)
    agg = (aggp_ref[0:NP8] + aggp_ref[NP8:2 * NP8]) / deg
    out = agg + jnp.dot(h, rw_ref[...], preferred_element_type=jnp.float32) + cb_ref[...]
    mt = _group_fold(jnp.mean(out, axis=0, keepdims=True))
    c = out - mt
    vt = _group_fold(jnp.mean(c * c, axis=0, keepdims=True))
    out = c * lax.rsqrt(vt + 1e-5) * gm_ref[...] + bt_ref[...]
    out_ref[...] = jnp.maximum(out, 0.0) + h


def _node_tc(aggp, degp, h, rwb, cbt, gmt, btt):
    return pl.pallas_call(
        _node_body,
        out_shape=jax.ShapeDtypeStruct((NP8, 128), jnp.float32),
    )(aggp, degp, h, rwb, cbt, gmt, btt)


def _pool_body(h_ref, batch_ref, w1_ref, b1_ref, w2_ref, b2_ref, w3_ref, b3_ref,
               out_ref):
    gid = lax.broadcasted_iota(jnp.int32, (NUM_GRAPHS, NP8), 0)
    sums = jnp.zeros((NUM_GRAPHS, HID), jnp.float32)
    cnt = jnp.zeros((NUM_GRAPHS, 1), jnp.float32)
    for m in range(8):
        onehot = jnp.where(batch_ref[m:m + 1, :] == gid, 1.0, 0.0)
        sums = sums + jnp.dot(onehot, h_ref[:, m * 16:(m + 1) * 16],
                              preferred_element_type=jnp.float32)
        cnt = cnt + jnp.sum(onehot, axis=1, keepdims=True)
    g = sums / jnp.maximum(cnt, 1.0)
    g = jnp.maximum(jnp.dot(g, w1_ref[...], preferred_element_type=jnp.float32)
                    + b1_ref[...], 0.0)
    g = jnp.maximum(jnp.dot(g, w2_ref[...], preferred_element_type=jnp.float32)
                    + b2_ref[...], 0.0)
    out_ref[...] = jnp.dot(g, w3_ref[...], preferred_element_type=jnp.float32) + b3_ref[...]


def _pool_tc(h, batch_pk, w1, b1, w2, b2, w3, b3):
    return pl.pallas_call(
        _pool_body,
        out_shape=jax.ShapeDtypeStruct((NUM_GRAPHS, 1), jnp.float32),
    )(h, batch_pk, w1, b1, w2, b2, w3, b3)


def kernel(x, edge_index, batch, edge_attr, nnW1, nnb1, nnW2, nnb2, rootW, convb,
           bn_gamma, bn_beta, mlpW1, mlpb1, mlpW2, mlpb2, mlpW3, mlpb3):
    src = edge_index[0].astype(jnp.int32)
    dst = edge_index[1].astype(jnp.int32)
    pad = E_PAD - E
    src2d = jnp.concatenate([src, jnp.zeros((pad,), jnp.int32)]).reshape(E_PAD // CHUNK, CHUNK)
    dst2d = jnp.concatenate([dst, jnp.full((pad,), N, jnp.int32)]).reshape(E_PAD // CHUNK, CHUNK)
    ea_pk = jnp.concatenate([edge_attr, jnp.zeros((pad, D), jnp.float32)],
                            axis=0).reshape(EP8, 128)
    x_pk = x.reshape(NP8, 128)
    batch_pk = batch.astype(jnp.int32).reshape(NP8, 8).T

    eye8 = jnp.eye(8, dtype=jnp.float32)
    t_all = nnW2.reshape(NUM_LAYERS, HID, D, HID)
    bdt = jnp.einsum('kjio,lp->klijpo', t_all, eye8).reshape(NUM_LAYERS, 128, 2048)
    rcat = jnp.einsum('aj,lp,o->lajpo', jnp.eye(16, dtype=jnp.float32), eye8,
                      jnp.ones((16,), jnp.float32)).reshape(128, 2048)
    w1b = jnp.einsum('kio,lp->klipo', nnW1, eye8).reshape(NUM_LAYERS, 128, 128)
    b2b = jnp.einsum('kio,lp->klipo', nnb2.reshape(NUM_LAYERS, D, HID),
                     eye8).reshape(NUM_LAYERS, 128, 128)
    rwb = jnp.einsum('kio,lp->klipo', rootW, eye8).reshape(NUM_LAYERS, 128, 128)
    b1t = jnp.tile(nnb1, (1, 8)).reshape(NUM_LAYERS, 1, 128)
    cbt = jnp.tile(convb, (1, 8)).reshape(NUM_LAYERS, 1, 128)
    gmt = jnp.tile(bn_gamma, (1, 8)).reshape(NUM_LAYERS, 1, 128)
    btt = jnp.tile(bn_beta, (1, 8)).reshape(NUM_LAYERS, 1, 128)

    h = x_pk
    degp = None
    for k in range(NUM_LAYERS):
        hs = _gather_sc()(h, src2d)
        msg = _bilinear_tc(hs, ea_pk, w1b[k], b1t[k], bdt[k], rcat, b2b[k])
        if k == 0:
            aggp, degp = _scatter_deg_sc()(msg, dst2d)
        else:
            aggp = _scatter_sc()(msg, dst2d)
        h = _node_tc(aggp, degp, h, rwb[k], cbt[k], gmt[k], btt[k])

    return _pool_tc(h, batch_pk, mlpW1, mlpb1.reshape(1, HID), mlpW2,
                    mlpb2.reshape(1, HID), mlpW3, mlpb3.reshape(1, 1))
